# baseline (speedup 1.0000x reference)
import jax
import jax.numpy as jnp
from jax import lax
from jax.experimental import pallas as pl
from jax.experimental.pallas import tpu as pltpu

_B = 4096
_H = 2048
_V = 101
_VP = 128
_OUT = 5
_OUTP = 128
_BB = 512
_NB = _B // _BB
_CHUNK = 128
_CPK = _H // _CHUNK
_NPE = 3 * _CPK


def _fused_kernel(embed_ref, w0_ref, b0_ref, w1_ref, b1_ref, w2_ref, b2_ref,
                  w3_ref, b3_ref, w4_ref, b4_ref, trip_ref, lbl_ref,
                  pred_ref, loss_ref, pe_ref):
    i = pl.program_id(0)

    @pl.when(i < _NPE)
    def _pe_phase():
        k = i // _CPK
        c = lax.rem(i, _CPK)
        part = jnp.dot(embed_ref[:, pl.ds(c * _CHUNK, _CHUNK)], w0_ref[...],
                       preferred_element_type=jnp.float32)
        row = k * _VP

        @pl.when(c == 0)
        def _():
            pe_ref[pl.ds(row, _VP), :] = part

        @pl.when(c != 0)
        def _():
            pe_ref[pl.ds(row, _VP), :] += part

    @pl.when(i >= _NPE)
    def _mlp_phase():
        trips = trip_ref[...]
        col = lax.broadcasted_iota(jnp.int32, (_BB, _VP), 1)
        oh = jnp.concatenate(
            [(col == trips[:, k:k + 1]).astype(jnp.float32)
             for k in range(3)], axis=1)
        h = jnp.dot(oh, pe_ref[...], preferred_element_type=jnp.float32)
        h = jnp.maximum(h + b0_ref[...], 0.0)
        h = jnp.maximum(
            jnp.dot(h, w1_ref[...], preferred_element_type=jnp.float32)
            + b1_ref[...], 0.0)
        h = jnp.maximum(
            jnp.dot(h, w2_ref[...], preferred_element_type=jnp.float32)
            + b2_ref[...], 0.0)
        h = jnp.maximum(
            jnp.dot(h, w3_ref[...], preferred_element_type=jnp.float32)
            + b3_ref[...], 0.0)
        pred = (jnp.dot(h, w4_ref[...], preferred_element_type=jnp.float32)
                + b4_ref[...])
        pred_ref[...] = pred

        ocol = lax.broadcasted_iota(jnp.int32, (_BB, _OUTP), 1)
        pm = jnp.where(ocol < _OUT, pred, -jnp.inf)
        m = jnp.max(pm, axis=1, keepdims=True)
        lse = m + jnp.log(jnp.sum(jnp.exp(pm - m), axis=1, keepdims=True))
        sel = (ocol == lbl_ref[...]).astype(jnp.float32)
        nll_sum = jnp.sum(sel * (lse - pred))

        @pl.when(i == _NPE)
        def _():
            loss_ref[0, 0] = 0.0

        loss_ref[0, 0] += nll_sum / _B


def kernel(embed, W0, b0, W1, b1, W2, b2, W3, b3, W4, b4, triples, labels):
    embed_p = jnp.pad(embed, ((0, _VP - _V), (0, 0)))
    w4p = jnp.pad(W4, ((0, 0), (0, _OUTP - _OUT)))
    b4p = jnp.pad(b4, (0, _OUTP - _OUT)).reshape(1, _OUTP)
    lbl2 = labels.reshape(_B, 1).astype(jnp.int32)

    const = lambda i: (0, 0)
    batch = lambda i: (jnp.maximum(i - _NPE, 0), 0)
    pred_p, loss = pl.pallas_call(
        _fused_kernel,
        grid=(_NPE + _NB,),
        in_specs=[
            pl.BlockSpec((_VP, _H), const),
            pl.BlockSpec((_CHUNK, _H),
                         lambda i: (jnp.minimum(i, _NPE - 1), 0)),
            pl.BlockSpec((1, _H), const),
            pl.BlockSpec((_H, _H), const),
            pl.BlockSpec((1, _H), const),
            pl.BlockSpec((_H, _H), const),
            pl.BlockSpec((1, _H), const),
            pl.BlockSpec((_H, _H), const),
            pl.BlockSpec((1, _H), const),
            pl.BlockSpec((_H, _OUTP), const),
            pl.BlockSpec((1, _OUTP), const),
            pl.BlockSpec((_BB, 3), batch),
            pl.BlockSpec((_BB, 1), batch),
        ],
        out_specs=[
            pl.BlockSpec((_BB, _OUTP), batch),
            pl.BlockSpec((1, 1), const, memory_space=pltpu.SMEM),
        ],
        out_shape=[
            jax.ShapeDtypeStruct((_B, _OUTP), jnp.float32),
            jax.ShapeDtypeStruct((1, 1), jnp.float32),
        ],
        scratch_shapes=[pltpu.VMEM((3 * _VP, _H), jnp.float32)],
        compiler_params=pltpu.CompilerParams(
            vmem_limit_bytes=128 * 1024 * 1024),
    )(embed_p, W0, b0.reshape(1, _H), W1, b1.reshape(1, _H),
      W2, b2.reshape(1, _H), W3, b3.reshape(1, _H), w4p, b4p,
      triples.astype(jnp.int32), lbl2)

    return pred_p[:, :_OUT], loss.reshape(())


# speedup vs baseline: 1.1143x; 1.1143x over previous
import jax
import jax.numpy as jnp
from jax import lax
from jax.experimental import pallas as pl
from jax.experimental.pallas import tpu as pltpu

_B = 4096
_H = 2048
_V = 101
_VP = 128
_OUT = 5
_OUTP = 128
_BB = 512
_NB = _B // _BB


def _pe_kernel(embed_ref, w0_ref, pe_ref):
    pe_ref[...] = jnp.dot(embed_ref[...], w0_ref[...],
                          preferred_element_type=jnp.float32)


def _mlp_kernel(pe_ref, b0_ref, w1_ref, b1_ref, w2_ref, b2_ref,
                w3_ref, b3_ref, w4_ref, b4_ref, trip_ref, lbl_ref,
                pred_ref, loss_ref):
    trips = trip_ref[...]
    col = lax.broadcasted_iota(jnp.int32, (_BB, _VP), 1)
    oh = jnp.concatenate(
        [(col == trips[:, k:k + 1]).astype(jnp.float32) for k in range(3)],
        axis=1)
    h = jnp.dot(oh, pe_ref[...], preferred_element_type=jnp.float32)
    h = jnp.maximum(h + b0_ref[...], 0.0)
    h = jnp.maximum(
        jnp.dot(h, w1_ref[...], preferred_element_type=jnp.float32)
        + b1_ref[...], 0.0)
    h = jnp.maximum(
        jnp.dot(h, w2_ref[...], preferred_element_type=jnp.float32)
        + b2_ref[...], 0.0)
    h = jnp.maximum(
        jnp.dot(h, w3_ref[...], preferred_element_type=jnp.float32)
        + b3_ref[...], 0.0)
    pred = (jnp.dot(h, w4_ref[...], preferred_element_type=jnp.float32)
            + b4_ref[...])
    pred_ref[...] = pred

    ocol = lax.broadcasted_iota(jnp.int32, (_BB, _OUTP), 1)
    pm = jnp.where(ocol < _OUT, pred, -jnp.inf)
    m = jnp.max(pm, axis=1, keepdims=True)
    lse = m + jnp.log(jnp.sum(jnp.exp(pm - m), axis=1, keepdims=True))
    sel = (ocol == lbl_ref[...]).astype(jnp.float32)
    nll_sum = jnp.sum(sel * (lse - pred))
    loss_ref[0, 0, 0] = nll_sum


def kernel(embed, W0, b0, W1, b1, W2, b2, W3, b3, W4, b4, triples, labels):
    embed_p = jnp.pad(embed, ((0, _VP - _V), (0, 0)))
    pe = pl.pallas_call(
        _pe_kernel,
        grid=(3,),
        in_specs=[
            pl.BlockSpec((_VP, _H), lambda k: (0, 0)),
            pl.BlockSpec((_H, _H), lambda k: (k, 0)),
        ],
        out_specs=pl.BlockSpec((_VP, _H), lambda k: (k, 0)),
        out_shape=jax.ShapeDtypeStruct((3 * _VP, _H), jnp.float32),
    )(embed_p, W0)

    w4p = jnp.pad(W4, ((0, 0), (0, _OUTP - _OUT)))
    b4p = jnp.pad(b4, (0, _OUTP - _OUT)).reshape(1, _OUTP)
    lbl2 = labels.reshape(_B, 1).astype(jnp.int32)

    const = lambda i: (0, 0)
    pred_p, nll_part = pl.pallas_call(
        _mlp_kernel,
        grid=(_NB,),
        in_specs=[
            pl.BlockSpec((3 * _VP, _H), const),
            pl.BlockSpec((1, _H), const),
            pl.BlockSpec((_H, _H), const),
            pl.BlockSpec((1, _H), const),
            pl.BlockSpec((_H, _H), const),
            pl.BlockSpec((1, _H), const),
            pl.BlockSpec((_H, _H), const),
            pl.BlockSpec((1, _H), const),
            pl.BlockSpec((_H, _OUTP), const),
            pl.BlockSpec((1, _OUTP), const),
            pl.BlockSpec((_BB, 3), lambda i: (i, 0)),
            pl.BlockSpec((_BB, 1), lambda i: (i, 0)),
        ],
        out_specs=[
            pl.BlockSpec((_BB, _OUTP), lambda i: (i, 0)),
            pl.BlockSpec((1, 1, 1), lambda i: (i, 0, 0), memory_space=pltpu.SMEM),
        ],
        out_shape=[
            jax.ShapeDtypeStruct((_B, _OUTP), jnp.float32),
            jax.ShapeDtypeStruct((_NB, 1, 1), jnp.float32),
        ],
        compiler_params=pltpu.CompilerParams(
            vmem_limit_bytes=128 * 1024 * 1024),
    )(pe, b0.reshape(1, _H), W1, b1.reshape(1, _H), W2, b2.reshape(1, _H),
      W3, b3.reshape(1, _H), w4p, b4p, triples.astype(jnp.int32), lbl2)

    return pred_p[:, :_OUT], jnp.sum(nll_part) / _B


# speedup vs baseline: 1.1359x; 1.0194x over previous
import jax
import jax.numpy as jnp
from jax import lax
from jax.experimental import pallas as pl
from jax.experimental.pallas import tpu as pltpu

_B = 4096
_H = 2048
_V = 101
_VP = 128
_OUT = 5
_OUTP = 128
_BB = 512
_NB = _B // _BB


def _pe_kernel(embed_ref, w0_ref, pe_ref):
    pe_ref[...] = jnp.dot(embed_ref[...], w0_ref[...],
                          preferred_element_type=jnp.float32)


def _mlp_kernel(pe_ref, b0_ref, w1_ref, b1_ref, w2_ref, b2_ref,
                w3_ref, b3_ref, w4_ref, b4_ref, trip_ref, lbl_ref,
                pred_ref, loss_ref):
    trips = trip_ref[...]
    col = lax.broadcasted_iota(jnp.int32, (_BB, _VP), 1)
    oh = jnp.concatenate(
        [(col == trips[:, k:k + 1]).astype(jnp.float32) for k in range(3)],
        axis=1)
    h = jnp.dot(oh, pe_ref[...], preferred_element_type=jnp.float32)
    h = jnp.maximum(h + b0_ref[...], 0.0)
    h = jnp.maximum(
        jnp.dot(h, w1_ref[...], preferred_element_type=jnp.float32)
        + b1_ref[...], 0.0)
    h = jnp.maximum(
        jnp.dot(h, w2_ref[...], preferred_element_type=jnp.float32)
        + b2_ref[...], 0.0)
    h = jnp.maximum(
        jnp.dot(h, w3_ref[...], preferred_element_type=jnp.float32)
        + b3_ref[...], 0.0)
    pred = (jnp.dot(h, w4_ref[...], preferred_element_type=jnp.float32)
            + b4_ref[...])
    pred_ref[...] = pred

    ocol = lax.broadcasted_iota(jnp.int32, (_BB, _OUTP), 1)
    pm = jnp.where(ocol < _OUT, pred, -jnp.inf)
    m = jnp.max(pm, axis=1, keepdims=True)
    lse = m + jnp.log(jnp.sum(jnp.exp(pm - m), axis=1, keepdims=True))
    sel = (ocol == lbl_ref[...]).astype(jnp.float32)
    nll_sum = jnp.sum(sel * (lse - pred))

    @pl.when(pl.program_id(0) == 0)
    def _():
        loss_ref[0, 0] = 0.0

    loss_ref[0, 0] += nll_sum / _B


def kernel(embed, W0, b0, W1, b1, W2, b2, W3, b3, W4, b4, triples, labels):
    embed_p = jnp.pad(embed, ((0, _VP - _V), (0, 0)))
    pe = pl.pallas_call(
        _pe_kernel,
        grid=(3,),
        in_specs=[
            pl.BlockSpec((_VP, _H), lambda k: (0, 0)),
            pl.BlockSpec((_H, _H), lambda k: (k, 0)),
        ],
        out_specs=pl.BlockSpec((_VP, _H), lambda k: (k, 0)),
        out_shape=jax.ShapeDtypeStruct((3 * _VP, _H), jnp.float32),
    )(embed_p, W0)

    w4p = jnp.pad(W4, ((0, 0), (0, _OUTP - _OUT)))
    b4p = jnp.pad(b4, (0, _OUTP - _OUT)).reshape(1, _OUTP)
    lbl2 = labels.reshape(_B, 1).astype(jnp.int32)

    const = lambda i: (0, 0)
    pred_p, loss = pl.pallas_call(
        _mlp_kernel,
        grid=(_NB,),
        in_specs=[
            pl.BlockSpec((3 * _VP, _H), const),
            pl.BlockSpec((1, _H), const),
            pl.BlockSpec((_H, _H), const),
            pl.BlockSpec((1, _H), const),
            pl.BlockSpec((_H, _H), const),
            pl.BlockSpec((1, _H), const),
            pl.BlockSpec((_H, _H), const),
            pl.BlockSpec((1, _H), const),
            pl.BlockSpec((_H, _OUTP), const),
            pl.BlockSpec((1, _OUTP), const),
            pl.BlockSpec((_BB, 3), lambda i: (i, 0)),
            pl.BlockSpec((_BB, 1), lambda i: (i, 0)),
        ],
        out_specs=[
            pl.BlockSpec((_BB, _OUTP), lambda i: (i, 0)),
            pl.BlockSpec((1, 1), const, memory_space=pltpu.SMEM),
        ],
        out_shape=[
            jax.ShapeDtypeStruct((_B, _OUTP), jnp.float32),
            jax.ShapeDtypeStruct((1, 1), jnp.float32),
        ],
        compiler_params=pltpu.CompilerParams(
            vmem_limit_bytes=128 * 1024 * 1024),
    )(pe, b0.reshape(1, _H), W1, b1.reshape(1, _H), W2, b2.reshape(1, _H),
      W3, b3.reshape(1, _H), w4p, b4p, triples.astype(jnp.int32), lbl2)

    return pred_p[:, :_OUT], loss.reshape(())


# speedup vs baseline: 1.1500x; 1.0124x over previous
import jax
import jax.numpy as jnp
from jax import lax
from jax.experimental import pallas as pl
from jax.experimental.pallas import tpu as pltpu

_B = 4096
_H = 2048
_V = 101
_VP = 128
_OUT = 5
_BB = 512
_NB = _B // _BB


def _pe_kernel(embed_ref, w0_ref, pe_ref):
    pe_ref[...] = jnp.dot(embed_ref[...], w0_ref[...],
                          preferred_element_type=jnp.float32)


def _mlp_kernel(pe_ref, b0_ref, w1_ref, b1_ref, w2_ref, b2_ref,
                w3_ref, b3_ref, w4_ref, b4_ref, trip_ref, lbl_ref,
                pred_ref, loss_ref, nll_ref):
    trips = trip_ref[...]
    col = lax.broadcasted_iota(jnp.int32, (_BB, _VP), 1)
    oh = jnp.concatenate(
        [(col == trips[:, k:k + 1]).astype(jnp.float32) for k in range(3)],
        axis=1)
    h = jnp.dot(oh, pe_ref[...], preferred_element_type=jnp.float32)
    h = jnp.maximum(h + b0_ref[...], 0.0)
    h = jnp.maximum(
        jnp.dot(h, w1_ref[...], preferred_element_type=jnp.float32)
        + b1_ref[...], 0.0)
    h = jnp.maximum(
        jnp.dot(h, w2_ref[...], preferred_element_type=jnp.float32)
        + b2_ref[...], 0.0)
    h = jnp.maximum(
        jnp.dot(h, w3_ref[...], preferred_element_type=jnp.float32)
        + b3_ref[...], 0.0)
    pred = (jnp.dot(h, w4_ref[...], preferred_element_type=jnp.float32)
            + b4_ref[...])
    pred_ref[...] = pred

    ocol = lax.broadcasted_iota(jnp.int32, (_BB, _OUT), 1)
    m = jnp.max(pred, axis=1, keepdims=True)
    lse = m + jnp.log(jnp.sum(jnp.exp(pred - m), axis=1, keepdims=True))
    sel = (ocol == lbl_ref[...]).astype(jnp.float32)
    i = pl.program_id(0)

    @pl.when(i == 0)
    def _():
        nll_ref[...] = jnp.zeros((_BB, _OUT), jnp.float32)

    nll_ref[...] += sel * (lse - pred)

    @pl.when(i == _NB - 1)
    def _():
        loss_ref[0, 0] = jnp.sum(nll_ref[...]) / _B


def kernel(embed, W0, b0, W1, b1, W2, b2, W3, b3, W4, b4, triples, labels):
    embed_p = jnp.pad(embed, ((0, _VP - _V), (0, 0)))
    pe = pl.pallas_call(
        _pe_kernel,
        grid=(3,),
        in_specs=[
            pl.BlockSpec((_VP, _H), lambda k: (0, 0)),
            pl.BlockSpec((_H, _H), lambda k: (k, 0)),
        ],
        out_specs=pl.BlockSpec((_VP, _H), lambda k: (k, 0)),
        out_shape=jax.ShapeDtypeStruct((3 * _VP, _H), jnp.float32),
    )(embed_p, W0)

    lbl2 = labels.reshape(_B, 1).astype(jnp.int32)

    const = lambda i: (0, 0)
    pred_p, loss = pl.pallas_call(
        _mlp_kernel,
        grid=(_NB,),
        in_specs=[
            pl.BlockSpec((3 * _VP, _H), const),
            pl.BlockSpec((1, _H), const),
            pl.BlockSpec((_H, _H), const),
            pl.BlockSpec((1, _H), const),
            pl.BlockSpec((_H, _H), const),
            pl.BlockSpec((1, _H), const),
            pl.BlockSpec((_H, _H), const),
            pl.BlockSpec((1, _H), const),
            pl.BlockSpec((_H, _OUT), const),
            pl.BlockSpec((1, _OUT), const),
            pl.BlockSpec((_BB, 3), lambda i: (i, 0)),
            pl.BlockSpec((_BB, 1), lambda i: (i, 0)),
        ],
        out_specs=[
            pl.BlockSpec((_BB, _OUT), lambda i: (i, 0)),
            pl.BlockSpec((1, 1), const, memory_space=pltpu.SMEM),
        ],
        out_shape=[
            jax.ShapeDtypeStruct((_B, _OUT), jnp.float32),
            jax.ShapeDtypeStruct((1, 1), jnp.float32),
        ],
        scratch_shapes=[pltpu.VMEM((_BB, _OUT), jnp.float32)],
        compiler_params=pltpu.CompilerParams(
            vmem_limit_bytes=128 * 1024 * 1024),
    )(pe, b0.reshape(1, _H), W1, b1.reshape(1, _H), W2, b2.reshape(1, _H),
      W3, b3.reshape(1, _H), W4, b4.reshape(1, _OUT),
      triples.astype(jnp.int32), lbl2)

    return pred_p, loss.reshape(())
